# baseline (speedup 1.0000x reference)
import functools

import jax
import jax.numpy as jnp
from jax import lax
from jax.experimental import pallas as pl
from jax.experimental.pallas import tpu as pltpu
from jax.experimental.pallas import tpu_sc as plsc

NUM_TABLES = 26
VOCAB = 100000
DIM = 32
BATCH = 4096

_C = 128


def _make_sc_gather():
    info = plsc.get_sparse_core_info()
    nw = info.num_cores * info.num_subcores
    total = NUM_TABLES * BATCH
    chunks_per_w = total // (_C * nw)
    assert total == chunks_per_w * _C * nw

    mesh = plsc.VectorSubcoreMesh(core_axis_name="c", subcore_axis_name="s")

    @functools.partial(
        pl.kernel,
        mesh=mesh,
        compiler_params=pltpu.CompilerParams(use_tc_tiling_on_sc=False),
        out_type=jax.ShapeDtypeStruct((total, DIM), jnp.float32),
        scratch_types=[
            pltpu.VMEM((chunks_per_w, _C), jnp.int32),
            pltpu.VMEM((_C,), jnp.int32),
            pltpu.VMEM((_C, DIM), jnp.float32),
            pltpu.SemaphoreType.DMA,
        ],
    )
    def sc_gather(tables_hbm, idx_hbm, out_hbm, idx_v, adj_v, rows_v, sem):
        wid = lax.axis_index("s") * info.num_cores + lax.axis_index("c")
        pltpu.sync_copy(idx_hbm.at[wid], idx_v)

        def chunk(j, carry):
            g = wid * chunks_per_w + j
            table = g // (BATCH // _C)
            off = table * VOCAB
            for i in range(_C // 16):
                adj_v[pl.ds(i * 16, 16)] = idx_v[j, pl.ds(i * 16, 16)] + off
            pltpu.async_copy(tables_hbm.at[adj_v], rows_v, sem).wait()
            pltpu.sync_copy(rows_v, out_hbm.at[pl.ds(g * _C, _C)])
            return carry

        lax.fori_loop(0, chunks_per_w, chunk, 0)

    return sc_gather, nw, chunks_per_w


def kernel(tables, indices, lengths):
    del lengths
    sc_gather, nw, chunks_per_w = _make_sc_gather()
    flat_tables = tables.reshape(NUM_TABLES * VOCAB, DIM)
    idx = indices.astype(jnp.int32).reshape(nw, chunks_per_w, _C)
    out = sc_gather(flat_tables, idx)
    return out.reshape(NUM_TABLES, BATCH, DIM)


# speedup vs baseline: 1.0170x; 1.0170x over previous
import functools

import jax
import jax.numpy as jnp
from jax import lax
from jax.experimental import pallas as pl
from jax.experimental.pallas import tpu as pltpu
from jax.experimental.pallas import tpu_sc as plsc

NUM_TABLES = 26
VOCAB = 100000
DIM = 32
BATCH = 4096

_C = 128


def _make_sc_gather():
    info = plsc.get_sparse_core_info()
    nw = info.num_cores * info.num_subcores
    total = NUM_TABLES * BATCH
    chunks_per_w = total // (_C * nw)
    assert total == chunks_per_w * _C * nw

    mesh = plsc.VectorSubcoreMesh(core_axis_name="c", subcore_axis_name="s")

    @functools.partial(
        pl.kernel,
        mesh=mesh,
        compiler_params=pltpu.CompilerParams(use_tc_tiling_on_sc=False),
        out_type=jax.ShapeDtypeStruct((total, DIM), jnp.float32),
        scratch_types=[
            pltpu.VMEM((chunks_per_w, _C), jnp.int32),
            pltpu.VMEM((chunks_per_w, _C), jnp.int32),
            pltpu.VMEM((chunks_per_w * _C, DIM), jnp.float32),
            pltpu.SemaphoreType.DMA,
        ],
    )
    def sc_gather(tables_hbm, idx_hbm, out_hbm, idx_v, adj_v, rows_v, sem):
        wid = lax.axis_index("s") * info.num_cores + lax.axis_index("c")
        pltpu.sync_copy(idx_hbm.at[wid], idx_v)

        def fire(j, carry):
            g = wid * chunks_per_w + j
            table = g // (BATCH // _C)
            off = table * VOCAB
            for i in range(_C // 16):
                adj_v[j, pl.ds(i * 16, 16)] = idx_v[j, pl.ds(i * 16, 16)] + off
            pltpu.async_copy(
                tables_hbm.at[adj_v.at[j]], rows_v.at[pl.ds(j * _C, _C)], sem
            )
            return carry

        lax.fori_loop(0, chunks_per_w, fire, 0)

        def drain(j, carry):
            pltpu.make_async_copy(
                tables_hbm.at[pl.ds(0, _C)], rows_v.at[pl.ds(j * _C, _C)], sem
            ).wait()
            return carry

        lax.fori_loop(0, chunks_per_w, drain, 0)
        pltpu.sync_copy(rows_v, out_hbm.at[pl.ds(wid * chunks_per_w * _C, chunks_per_w * _C)])

    return sc_gather, nw, chunks_per_w


def kernel(tables, indices, lengths):
    del lengths
    sc_gather, nw, chunks_per_w = _make_sc_gather()
    flat_tables = tables.reshape(NUM_TABLES * VOCAB, DIM)
    idx = indices.astype(jnp.int32).reshape(nw, chunks_per_w, _C)
    out = sc_gather(flat_tables, idx)
    return out.reshape(NUM_TABLES, BATCH, DIM)
